# baseline (device time: 33933 ns/iter reference)
import jax
import jax.numpy as jnp
from jax import lax
from jax.experimental import pallas as pl
from jax.experimental.pallas import tpu as pltpu

NC = 4


def kernel(dy, W):
    m, k = dy.shape
    d, _ = W.shape
    HALF = m // 2
    Q = m // 4
    CW = d // NC

    def body(dy_any, w_any, out_ref,
             dyv, dybf, wv, wbf, pn, po, rsb,
             dy_sem, w_sems, rs_s, rs_r, ax_s, ax_r, ay_s, ay_r, b_s, b_r):
        my_x = lax.axis_index("x")
        my_y = lax.axis_index("y")
        x_nb = (1 - my_x, my_y)
        y_nb = (my_x, 1 - my_y)

        dy_dma = pltpu.make_async_copy(
            dy_any.at[pl.ds(my_y * HALF, HALF), :], dyv, dy_sem
        )
        dy_dma.start()

        def w_dma(c):
            return pltpu.make_async_copy(
                w_any.at[pl.ds(c * CW, CW), :], wv.at[c % 2], w_sems.at[c % 2]
            )

        w_dma(0).start()
        w_dma(1).start()

        barrier_sem = pltpu.get_barrier_semaphore()
        pl.semaphore_signal(
            barrier_sem, inc=1,
            device_id=x_nb, device_id_type=pl.DeviceIdType.MESH,
        )
        pl.semaphore_signal(
            barrier_sem, inc=1,
            device_id=y_nb, device_id_type=pl.DeviceIdType.MESH,
        )
        pl.semaphore_wait(barrier_sem, 2)

        dy_dma.wait()
        dybf[...] = dyv[...].astype(jnp.bfloat16)

        my_off = my_y * HALF + my_x * Q
        xq_off = my_y * HALF + (1 - my_x) * Q

        def remote(src, dst, ssem, rsem, dev):
            return pltpu.make_async_remote_copy(
                src_ref=src, dst_ref=dst, send_sem=ssem, recv_sem=rsem,
                device_id=dev, device_id_type=pl.DeviceIdType.MESH,
            )

        def out_q(off, c):
            return out_ref.at[pl.ds(off, Q), pl.ds(c * CW, CW)]

        rs = {c: remote(pn.at[c], rsb.at[c], rs_s.at[c], rs_r.at[c], x_nb)
              for c in range(NC)}
        ax = {c: remote(out_q(my_off, c), out_q(my_off, c),
                        ax_s.at[c], ax_r.at[c], x_nb) for c in range(NC)}
        ay = {c: remote(out_q(my_off, c), out_q(my_off, c),
                        ay_s.at[c], ay_r.at[c], y_nb) for c in range(NC)}
        fw = {c: remote(out_q(xq_off, c), out_q(xq_off, c),
                        b_s.at[c], b_r.at[c], y_nb) for c in range(NC)}

        def compute_stage(c):
            w_dma(c).wait()
            wbf[c % 2] = wv[c % 2].astype(jnp.bfloat16)
            if c + 2 < NC:
                w_dma(c + 2).start()
            po[c] = lax.dot_general(
                dybf[...], wbf[c % 2], (((1,), (1,)), ((), ())),
                preferred_element_type=jnp.float32,
            )
            pn[c] = po[c, pl.ds((1 - my_x) * Q, Q), :].astype(jnp.bfloat16)
            rs[c].start()

        def rs_finish(c):
            rs[c].wait_recv()
            r32 = po[c, pl.ds(my_x * Q, Q), :] + rsb[c].astype(jnp.float32)
            out_ref[pl.ds(my_off, Q), pl.ds(c * CW, CW)] = (
                r32.astype(jnp.bfloat16)
            )
            ax[c].start()
            ay[c].start()

        def ab_finish(c):
            ax[c].wait_recv()
            fw[c].start()
            ay[c].wait_recv()

        def b_finish(c):
            fw[c].wait_recv()

        for c in range(NC):
            compute_stage(c)
            if c >= 1:
                rs_finish(c - 1)
            if c >= 2:
                ab_finish(c - 2)
            if c >= 3:
                b_finish(c - 3)
        rs_finish(NC - 1)
        ab_finish(NC - 2)
        b_finish(NC - 3)
        ab_finish(NC - 1)
        b_finish(NC - 2)
        b_finish(NC - 1)

        for c in range(NC):
            rs[c].wait_send()
            ax[c].wait_send()
            ay[c].wait_send()
            fw[c].wait_send()

    return pl.pallas_call(
        body,
        out_shape=jax.ShapeDtypeStruct((m, d), jnp.bfloat16),
        in_specs=[
            pl.BlockSpec(memory_space=pltpu.MemorySpace.HBM),
            pl.BlockSpec(memory_space=pltpu.MemorySpace.HBM),
        ],
        out_specs=pl.BlockSpec(memory_space=pltpu.VMEM),
        scratch_shapes=[
            pltpu.VMEM((HALF, k), jnp.float32),
            pltpu.VMEM((HALF, k), jnp.bfloat16),
            pltpu.VMEM((2, CW, k), jnp.float32),
            pltpu.VMEM((2, CW, k), jnp.bfloat16),
            pltpu.VMEM((NC, Q, CW), jnp.bfloat16),
            pltpu.VMEM((NC, HALF, CW), jnp.float32),
            pltpu.VMEM((NC, Q, CW), jnp.bfloat16),
            pltpu.SemaphoreType.DMA,
            pltpu.SemaphoreType.DMA((2,)),
            pltpu.SemaphoreType.DMA((NC,)),
            pltpu.SemaphoreType.DMA((NC,)),
            pltpu.SemaphoreType.DMA((NC,)),
            pltpu.SemaphoreType.DMA((NC,)),
            pltpu.SemaphoreType.DMA((NC,)),
            pltpu.SemaphoreType.DMA((NC,)),
            pltpu.SemaphoreType.DMA((NC,)),
            pltpu.SemaphoreType.DMA((NC,)),
        ],
        compiler_params=pltpu.CompilerParams(collective_id=0),
    )(dy, W)
